# baseline (speedup 1.0000x reference)
import functools

import jax
import jax.numpy as jnp
from jax import lax
from jax.experimental import pallas as pl
from jax.experimental.pallas import tpu as pltpu
from jax.experimental.pallas import tpu_sc as plsc

D = 128
K = 65536
BP = 16384
BS = 16384
C = 2048
NG = K // C


def _win(g, ptr, B):
    t0 = (g * C - ptr) % K
    in_win = (t0 < B) | (t0 > K - C)
    full_in = t0 <= B - C
    o = jnp.where(t0 < B, t0, t0 - K)
    oc = jnp.clip(o, 0, B - C)
    return t0, in_win, full_in, o, oc


def _qmap_p(g, pp, ps):
    _, _, full_in, _, _ = _win(g, pp[0], BP)
    return (0, jnp.where(full_in, 0, g))


def _qmap_s(g, pp, ps):
    _, _, full_in, _, _ = _win(g, ps[0], BS)
    return (0, jnp.where(full_in, 0, g))


def _body(pp_ref, ps_ref, qp_ref, qs_ref, kp_hbm, ks_hbm,
          out_ref, kbuf_p, kbuf_s, sem_p, sem_s):
    g = pl.program_id(0)
    c0 = g * C
    ptr_p = pp_ref[0]
    ptr_s = ps_ref[0]

    def issue(gg, ptr, B, khbm, kbuf, sem):
        t0 = (gg * C - ptr) % K
        in_win = (t0 < B) | (t0 > K - C)

        @pl.when(in_win & (gg < NG))
        def _():
            o = jnp.where(t0 < B, t0, t0 - K)
            oc = jnp.clip(o, 0, B - C)
            pltpu.make_async_copy(
                khbm.at[pl.ds(oc, C), :], kbuf.at[gg % 3], sem.at[gg % 3]
            ).start()

    @pl.when(g == 0)
    def _():
        issue(0, ptr_p, BP, kp_hbm, kbuf_p, sem_p)
        issue(0, ptr_s, BS, ks_hbm, kbuf_s, sem_s)
        issue(1, ptr_p, BP, kp_hbm, kbuf_p, sem_p)
        issue(1, ptr_s, BS, ks_hbm, kbuf_s, sem_s)

    issue(g + 2, ptr_p, BP, kp_hbm, kbuf_p, sem_p)
    issue(g + 2, ptr_s, BS, ks_hbm, kbuf_s, sem_s)

    def enqueue_half(ptr, B, khbm, kbuf, sem, q_ref, row0):
        t0, in_win, full_in, o, oc = _win(g, ptr, B)

        def normalized_t(s):
            t = s.T
            ssq = jnp.sum(t * t, axis=0, keepdims=True)
            return t * (1.0 / (jnp.sqrt(ssq) + 1e-12))

        def wait():
            pltpu.make_async_copy(
                khbm.at[pl.ds(oc, C), :], kbuf.at[g % 3], sem.at[g % 3]
            ).wait()

        @pl.when(full_in)
        def _():
            wait()
            out_ref[row0:row0 + D, :] = normalized_t(kbuf[g % 3])

        @pl.when(in_win & jnp.logical_not(full_in))
        def _():
            wait()
            resid = o - oc
            raw = kbuf[g % 3]
            s = lax.cond(
                resid == 0,
                lambda: raw,
                lambda: pltpu.roll(raw, -resid, 0),
            )
            sn = normalized_t(s)
            col = lax.broadcasted_iota(jnp.int32, (1, C), 1) + c0
            t = (col - ptr) % K
            out_ref[row0:row0 + D, :] = jnp.where(t < B, sn, q_ref[...])

        @pl.when(jnp.logical_not(in_win))
        def _():
            out_ref[row0:row0 + D, :] = q_ref[...]

    enqueue_half(ptr_p, BP, kp_hbm, kbuf_p, sem_p, qp_ref, 0)
    enqueue_half(ptr_s, BS, ks_hbm, kbuf_s, sem_s, qs_ref, D)


def _make_call(interpret=False):
    grid_spec = pltpu.PrefetchScalarGridSpec(
        num_scalar_prefetch=2,
        grid=(NG,),
        in_specs=[
            pl.BlockSpec((D, C), _qmap_p),
            pl.BlockSpec((D, C), _qmap_s),
            pl.BlockSpec(memory_space=pl.MemorySpace.ANY),
            pl.BlockSpec(memory_space=pl.MemorySpace.ANY),
        ],
        out_specs=pl.BlockSpec((2 * D, C), lambda g, pp, ps: (0, g)),
        scratch_shapes=[
            pltpu.VMEM((3, C, D), jnp.float32),
            pltpu.VMEM((3, C, D), jnp.float32),
            pltpu.SemaphoreType.DMA((3,)),
            pltpu.SemaphoreType.DMA((3,)),
        ],
    )
    return pl.pallas_call(
        _body,
        grid_spec=grid_spec,
        out_shape=jax.ShapeDtypeStruct((2 * D, K), jnp.float32),
        interpret=interpret,
    )


_SC_NC = 2
_SC_NS = 16
_SC_NW = _SC_NC * _SC_NS
_LROWS = K // 128 // _SC_NW


def _sc_labels_kernel():
    mesh = plsc.VectorSubcoreMesh(core_axis_name="c", subcore_axis_name="s")

    @functools.partial(
        pl.kernel,
        mesh=mesh,
        out_type=jax.ShapeDtypeStruct((K // 128, 128), jnp.int32),
        scratch_types=[
            pltpu.VMEM((_LROWS, 128), jnp.int32),
            pltpu.VMEM((_LROWS, 128), jnp.int32),
            pltpu.VMEM((16,), jnp.int32),
            pltpu.SemaphoreType.DMA,
        ],
    )
    def lab_kernel(src_hbm, ptr_hbm, out_hbm, idx_ref, val_ref, pvm, sem):
        wid = lax.axis_index("s") * _SC_NC + lax.axis_index("c")
        row0 = wid * _LROWS
        pltpu.sync_copy(ptr_hbm, pvm)
        p = pvm[...]

        def ibody(i, _):
            j = i // 8
            lane0 = (i - j * 8) * 16
            v = (row0 + j) * 128 + lane0 + lax.iota(jnp.int32, 16)
            t = lax.rem(v - p + K, K)
            idx_ref[j, pl.ds(lane0, 16)] = jnp.where(t < BS, t, BS + v)
            return 0

        lax.fori_loop(0, _LROWS * 8, ibody, 0)

        def gstart(j, _):
            pltpu.make_async_copy(
                src_hbm.at[idx_ref.at[j]], val_ref.at[j], sem).start()
            return 0

        lax.fori_loop(0, _LROWS, gstart, 0)

        def gwait(j, _):
            pltpu.make_async_copy(
                src_hbm.at[idx_ref.at[j]], val_ref.at[j], sem).wait()
            return 0

        lax.fori_loop(0, _LROWS, gwait, 0)

        pltpu.sync_copy(val_ref, out_hbm.at[pl.ds(row0, _LROWS), :])

    return lab_kernel


def kernel(queue_pcd, queue_seg, keys_pcd, keys_seg, k_labels, seg_labels,
           ptr_pcd, ptr_seg):
    pp = jnp.asarray(ptr_pcd, jnp.int32).reshape(1)
    ps = jnp.asarray(ptr_seg, jnp.int32).reshape(1)
    queues = _make_call()(pp, ps, queue_pcd, queue_seg, keys_pcd, keys_seg)
    lab_src = jnp.concatenate([seg_labels, k_labels])
    ps16 = jnp.broadcast_to(jnp.asarray(ptr_seg, jnp.int32), (16,))
    labels = _sc_labels_kernel()(lab_src, ps16).reshape(K)
    new_ptr_pcd = ((jnp.asarray(ptr_pcd, jnp.int32) + BP) % K).astype(jnp.int32)
    new_ptr_seg = ((jnp.asarray(ptr_seg, jnp.int32) + BS) % K).astype(jnp.int32)
    return (queues, labels, new_ptr_pcd, new_ptr_seg)


# speedup vs baseline: 1.0030x; 1.0030x over previous
import functools

import jax
import jax.numpy as jnp
from jax import lax
from jax.experimental import pallas as pl
from jax.experimental.pallas import tpu as pltpu
from jax.experimental.pallas import tpu_sc as plsc

D = 128
K = 65536
BP = 16384
BS = 16384
C = 2048
NG = K // C


def _win(g, ptr, B):
    t0 = (g * C - ptr) % K
    in_win = (t0 < B) | (t0 > K - C)
    full_in = t0 <= B - C
    o = jnp.where(t0 < B, t0, t0 - K)
    oc = jnp.clip(o, 0, B - C)
    return t0, in_win, full_in, o, oc


def _qmap_p(g, pp, ps):
    _, _, full_in, _, _ = _win(g, pp[0], BP)
    return (0, jnp.where(full_in, 0, g))


def _qmap_s(g, pp, ps):
    _, _, full_in, _, _ = _win(g, ps[0], BS)
    return (0, jnp.where(full_in, 0, g))


def _body(pp_ref, ps_ref, qp_ref, qs_ref, kp_hbm, ks_hbm,
          out_ref, kbuf_p, kbuf_s, sem_p, sem_s):
    g = pl.program_id(0)
    c0 = g * C
    ptr_p = pp_ref[0]
    ptr_s = ps_ref[0]

    def issue(gg, ptr, B, khbm, kbuf, sem):
        t0 = (gg * C - ptr) % K
        in_win = (t0 < B) | (t0 > K - C)

        @pl.when(in_win & (gg < NG))
        def _():
            o = jnp.where(t0 < B, t0, t0 - K)
            oc = jnp.clip(o, 0, B - C)
            pltpu.make_async_copy(
                khbm.at[pl.ds(oc, C), :], kbuf.at[gg % 3], sem.at[gg % 3]
            ).start()

    @pl.when(g == 0)
    def _():
        issue(0, ptr_p, BP, kp_hbm, kbuf_p, sem_p)
        issue(0, ptr_s, BS, ks_hbm, kbuf_s, sem_s)
        issue(1, ptr_p, BP, kp_hbm, kbuf_p, sem_p)
        issue(1, ptr_s, BS, ks_hbm, kbuf_s, sem_s)

    issue(g + 2, ptr_p, BP, kp_hbm, kbuf_p, sem_p)
    issue(g + 2, ptr_s, BS, ks_hbm, kbuf_s, sem_s)

    def enqueue_half(ptr, B, khbm, kbuf, sem, q_ref, row0):
        t0, in_win, full_in, o, oc = _win(g, ptr, B)

        def normalized_t(s):
            t = s.T
            ssq = jnp.sum(t * t, axis=0, keepdims=True)
            return t * (1.0 / (jnp.sqrt(ssq) + 1e-12))

        def wait():
            pltpu.make_async_copy(
                khbm.at[pl.ds(oc, C), :], kbuf.at[g % 3], sem.at[g % 3]
            ).wait()

        @pl.when(full_in)
        def _():
            wait()
            out_ref[row0:row0 + D, :] = normalized_t(kbuf[g % 3])

        @pl.when(in_win & jnp.logical_not(full_in))
        def _():
            wait()
            resid = o - oc
            raw = kbuf[g % 3]
            s = lax.cond(
                resid == 0,
                lambda: raw,
                lambda: pltpu.roll(raw, -resid, 0),
            )
            sn = normalized_t(s)
            col = lax.broadcasted_iota(jnp.int32, (1, C), 1) + c0
            t = (col - ptr) % K
            out_ref[row0:row0 + D, :] = jnp.where(t < B, sn, q_ref[...])

        @pl.when(jnp.logical_not(in_win))
        def _():
            out_ref[row0:row0 + D, :] = q_ref[...]

    enqueue_half(ptr_p, BP, kp_hbm, kbuf_p, sem_p, qp_ref, 0)
    enqueue_half(ptr_s, BS, ks_hbm, kbuf_s, sem_s, qs_ref, D)


def _make_call(interpret=False):
    grid_spec = pltpu.PrefetchScalarGridSpec(
        num_scalar_prefetch=2,
        grid=(NG,),
        in_specs=[
            pl.BlockSpec((D, C), _qmap_p),
            pl.BlockSpec((D, C), _qmap_s),
            pl.BlockSpec(memory_space=pl.MemorySpace.ANY),
            pl.BlockSpec(memory_space=pl.MemorySpace.ANY),
        ],
        out_specs=pl.BlockSpec((2 * D, C), lambda g, pp, ps: (0, g)),
        scratch_shapes=[
            pltpu.VMEM((3, C, D), jnp.float32),
            pltpu.VMEM((3, C, D), jnp.float32),
            pltpu.SemaphoreType.DMA((3,)),
            pltpu.SemaphoreType.DMA((3,)),
        ],
    )
    return pl.pallas_call(
        _body,
        grid_spec=grid_spec,
        out_shape=jax.ShapeDtypeStruct((2 * D, K), jnp.float32),
        interpret=interpret,
    )


_SC_NC = 2
_SC_NS = 16
_SC_NW = _SC_NC * _SC_NS
_LROWS = K // 128 // _SC_NW


def _sc_labels_kernel():
    mesh = plsc.VectorSubcoreMesh(core_axis_name="c", subcore_axis_name="s")

    @functools.partial(
        pl.kernel,
        mesh=mesh,
        out_type=jax.ShapeDtypeStruct((K // 128, 128), jnp.int32),
        scratch_types=[
            pltpu.VMEM((_LROWS, 128), jnp.int32),
            pltpu.VMEM((_LROWS, 128), jnp.int32),
            pltpu.VMEM((16,), jnp.int32),
            pltpu.SemaphoreType.DMA,
        ],
    )
    def lab_kernel(src_hbm, ptr_hbm, out_hbm, idx_ref, val_ref, pvm, sem):
        wid = lax.axis_index("s") * _SC_NC + lax.axis_index("c")
        row0 = wid * _LROWS
        pltpu.sync_copy(ptr_hbm, pvm)
        p = pvm[...]

        for j in range(_LROWS):
            for l8 in range(8):
                lane0 = l8 * 16
                v = (row0 + j) * 128 + lane0 + lax.iota(jnp.int32, 16)
                t = lax.rem(v - p + K, K)
                idx_ref[j, pl.ds(lane0, 16)] = jnp.where(t < BS, t, BS + v)
            pltpu.make_async_copy(
                src_hbm.at[idx_ref.at[j]], val_ref.at[j], sem).start()

        for j in range(_LROWS):
            pltpu.make_async_copy(
                src_hbm.at[idx_ref.at[j]], val_ref.at[j], sem).wait()

        pltpu.sync_copy(val_ref, out_hbm.at[pl.ds(row0, _LROWS), :])

    return lab_kernel


def kernel(queue_pcd, queue_seg, keys_pcd, keys_seg, k_labels, seg_labels,
           ptr_pcd, ptr_seg):
    pp = jnp.asarray(ptr_pcd, jnp.int32).reshape(1)
    ps = jnp.asarray(ptr_seg, jnp.int32).reshape(1)
    lab_src = jnp.concatenate([seg_labels, k_labels])
    ps16 = jnp.broadcast_to(jnp.asarray(ptr_seg, jnp.int32), (16,))
    labels = _sc_labels_kernel()(lab_src, ps16).reshape(K)
    queues = _make_call()(pp, ps, queue_pcd, queue_seg, keys_pcd, keys_seg)
    new_ptr_pcd = ((jnp.asarray(ptr_pcd, jnp.int32) + BP) % K).astype(jnp.int32)
    new_ptr_seg = ((jnp.asarray(ptr_seg, jnp.int32) + BS) % K).astype(jnp.int32)
    return (queues, labels, new_ptr_pcd, new_ptr_seg)


# speedup vs baseline: 1.0291x; 1.0260x over previous
import functools

import jax
import jax.numpy as jnp
from jax import lax
from jax.experimental import pallas as pl
from jax.experimental.pallas import tpu as pltpu
from jax.experimental.pallas import tpu_sc as plsc

D = 128
K = 65536
BP = 16384
BS = 16384
C = 2048
NG = K // C


def _win(g, ptr, B):
    t0 = (g * C - ptr) % K
    in_win = (t0 < B) | (t0 > K - C)
    full_in = t0 <= B - C
    o = jnp.where(t0 < B, t0, t0 - K)
    oc = jnp.clip(o, 0, B - C)
    return t0, in_win, full_in, o, oc


def _qmap_p(g, pp, ps):
    _, _, full_in, _, _ = _win(g, pp[0], BP)
    return (0, jnp.where(full_in, 0, g))


def _qmap_s(g, pp, ps):
    _, _, full_in, _, _ = _win(g, ps[0], BS)
    return (0, jnp.where(full_in, 0, g))


def _body(pp_ref, ps_ref, qp_ref, qs_ref, kp_hbm, ks_hbm,
          out_ref, kbuf_p, kbuf_s, sem_p, sem_s):
    g = pl.program_id(0)
    c0 = g * C
    ptr_p = pp_ref[0]
    ptr_s = ps_ref[0]

    def issue(gg, ptr, B, khbm, kbuf, sem):
        t0 = (gg * C - ptr) % K
        in_win = (t0 < B) | (t0 > K - C)

        @pl.when(in_win & (gg < NG))
        def _():
            o = jnp.where(t0 < B, t0, t0 - K)
            oc = jnp.clip(o, 0, B - C)
            pltpu.make_async_copy(
                khbm.at[pl.ds(oc, C), :], kbuf.at[gg % 3], sem.at[gg % 3]
            ).start()

    @pl.when(g == 0)
    def _():
        issue(0, ptr_p, BP, kp_hbm, kbuf_p, sem_p)
        issue(0, ptr_s, BS, ks_hbm, kbuf_s, sem_s)
        issue(1, ptr_p, BP, kp_hbm, kbuf_p, sem_p)
        issue(1, ptr_s, BS, ks_hbm, kbuf_s, sem_s)

    issue(g + 2, ptr_p, BP, kp_hbm, kbuf_p, sem_p)
    issue(g + 2, ptr_s, BS, ks_hbm, kbuf_s, sem_s)

    def enqueue_half(ptr, B, khbm, kbuf, sem, q_ref, row0):
        t0, in_win, full_in, o, oc = _win(g, ptr, B)

        def normalized_t(s):
            t = s.T
            ssq = jnp.sum(t * t, axis=0, keepdims=True)
            return t * (1.0 / (jnp.sqrt(ssq) + 1e-12))

        def wait():
            pltpu.make_async_copy(
                khbm.at[pl.ds(oc, C), :], kbuf.at[g % 3], sem.at[g % 3]
            ).wait()

        @pl.when(full_in)
        def _():
            wait()
            out_ref[row0:row0 + D, :] = normalized_t(kbuf[g % 3])

        @pl.when(in_win & jnp.logical_not(full_in))
        def _():
            wait()
            resid = o - oc
            raw = kbuf[g % 3]
            s = lax.cond(
                resid == 0,
                lambda: raw,
                lambda: pltpu.roll(raw, -resid, 0),
            )
            sn = normalized_t(s)
            col = lax.broadcasted_iota(jnp.int32, (1, C), 1) + c0
            t = (col - ptr) % K
            out_ref[row0:row0 + D, :] = jnp.where(t < B, sn, q_ref[...])

        @pl.when(jnp.logical_not(in_win))
        def _():
            out_ref[row0:row0 + D, :] = q_ref[...]

    enqueue_half(ptr_p, BP, kp_hbm, kbuf_p, sem_p, qp_ref, 0)
    enqueue_half(ptr_s, BS, ks_hbm, kbuf_s, sem_s, qs_ref, D)


def _make_call(interpret=False):
    grid_spec = pltpu.PrefetchScalarGridSpec(
        num_scalar_prefetch=2,
        grid=(NG,),
        in_specs=[
            pl.BlockSpec((D, C), _qmap_p),
            pl.BlockSpec((D, C), _qmap_s),
            pl.BlockSpec(memory_space=pl.MemorySpace.ANY),
            pl.BlockSpec(memory_space=pl.MemorySpace.ANY),
        ],
        out_specs=pl.BlockSpec((2 * D, C), lambda g, pp, ps: (0, g)),
        scratch_shapes=[
            pltpu.VMEM((3, C, D), jnp.float32),
            pltpu.VMEM((3, C, D), jnp.float32),
            pltpu.SemaphoreType.DMA((3,)),
            pltpu.SemaphoreType.DMA((3,)),
        ],
    )
    return pl.pallas_call(
        _body,
        grid_spec=grid_spec,
        out_shape=jax.ShapeDtypeStruct((2 * D, K), jnp.float32),
        interpret=interpret,
    )


_SC_NC = 1
_SC_NS = 16
_SC_NW = _SC_NC * _SC_NS
_LROWS = K // 128 // _SC_NW


def _sc_labels_kernel():
    mesh = plsc.VectorSubcoreMesh(core_axis_name="c", subcore_axis_name="s", num_cores=1)

    @functools.partial(
        pl.kernel,
        mesh=mesh,
        out_type=jax.ShapeDtypeStruct((K // 128, 128), jnp.int32),
        scratch_types=[
            pltpu.VMEM((_LROWS, 128), jnp.int32),
            pltpu.VMEM((_LROWS, 128), jnp.int32),
            pltpu.VMEM((16,), jnp.int32),
            pltpu.SemaphoreType.DMA,
        ],
    )
    def lab_kernel(src_hbm, ptr_hbm, out_hbm, idx_ref, val_ref, pvm, sem):
        wid = lax.axis_index("s") * _SC_NC + lax.axis_index("c")
        row0 = wid * _LROWS
        pltpu.sync_copy(ptr_hbm, pvm)
        p = pvm[...]

        for j in range(_LROWS):
            for l8 in range(8):
                lane0 = l8 * 16
                v = (row0 + j) * 128 + lane0 + lax.iota(jnp.int32, 16)
                t = lax.rem(v - p + K, K)
                idx_ref[j, pl.ds(lane0, 16)] = jnp.where(t < BS, t, BS + v)
            pltpu.make_async_copy(
                src_hbm.at[idx_ref.at[j]], val_ref.at[j], sem).start()

        for j in range(_LROWS):
            pltpu.make_async_copy(
                src_hbm.at[idx_ref.at[j]], val_ref.at[j], sem).wait()

        pltpu.sync_copy(val_ref, out_hbm.at[pl.ds(row0, _LROWS), :])

    return lab_kernel


def kernel(queue_pcd, queue_seg, keys_pcd, keys_seg, k_labels, seg_labels,
           ptr_pcd, ptr_seg):
    pp = jnp.asarray(ptr_pcd, jnp.int32).reshape(1)
    ps = jnp.asarray(ptr_seg, jnp.int32).reshape(1)
    lab_src = jnp.concatenate([seg_labels, k_labels])
    ps16 = jnp.broadcast_to(jnp.asarray(ptr_seg, jnp.int32), (16,))
    labels = _sc_labels_kernel()(lab_src, ps16).reshape(K)
    queues = _make_call()(pp, ps, queue_pcd, queue_seg, keys_pcd, keys_seg)
    new_ptr_pcd = ((jnp.asarray(ptr_pcd, jnp.int32) + BP) % K).astype(jnp.int32)
    new_ptr_seg = ((jnp.asarray(ptr_seg, jnp.int32) + BS) % K).astype(jnp.int32)
    return (queues, labels, new_ptr_pcd, new_ptr_seg)
